# Initial kernel scaffold; baseline (speedup 1.0000x reference)
import jax
import jax.numpy as jnp
from jax.experimental import pallas as pl


def kernel(x_tile, x_vertex, x_road, ei_tt, ei_tv, ei_vt, ei_tr, ei_rt, ei_vr, ei_rv, ei_vv, ei_rr, params):
    raise NotImplementedError("write your pallas kernel here")


# speedup vs baseline: 19.1167x; 19.1167x over previous
import functools
from typing import Any

import jax
import jax.numpy as jnp
from jax import lax
from jax.experimental import pallas as pl
from jax.experimental.pallas import tpu as pltpu
from jax.experimental.pallas import tpu_sc as plsc

N = 10000
E = 64000
ETOT = E + N
HID = 128
NHEAD = 2
CH = 64
RELS = ('tt', 'tv', 'vt', 'tr', 'rt', 'vr', 'rv', 'vv', 'rr')
TYPES = ('t', 'v', 'r')

NWORK = 32
BLK = 64
NCHUNK = 37
EPW = BLK * NCHUNK
EPAD = EPW * NWORK
ACC_ROWS = 10240
WACC = 144
GARBAGE = 10008
ZROWS = 64
NZCH = 10

SRC_RELS = {t: [r for r in RELS if r[0] == t] for t in TYPES}
DST_RELS = {t: [r for r in RELS if r[1] == t] for t in TYPES}
L_SLOT = {r: SRC_RELS[r[0]].index(r) for r in RELS}
R_SLOT = {r: 3 + DST_RELS[r[1]].index(r) for r in RELS}
TIDX = {'t': 0, 'v': 1, 'r': 2}
DST_REL_IDS = [[RELS.index(r) for r in DST_RELS[t]] for t in TYPES]

BR = 1000
GRID = N // BR


def _proj_in_body(xt, xv, xr, wt, bt, wv, bv, wr, br, out):
    t = pl.program_id(0)

    def mk(x, w, b):
        def f():
            out[...] = jnp.maximum(
                jnp.dot(x[...], w[...], preferred_element_type=jnp.float32) + b[...], 0.0)
        return f

    branches = [mk(xt, wt, bt), mk(xv, wv, bv), mk(xr, wr, br)]
    lax.switch(t, branches)


def _proj_in(xt, xv, xr, p):
    def w(name):
        return p[name]['W'], p[name]['b'].reshape(1, HID)
    wt, bt = w('in_tile')
    wv, bv = w('in_vertex')
    wr, br = w('in_road')
    row = lambda t, i: (i, 0)
    whole = lambda t, i: (0, 0)
    return pl.pallas_call(
        _proj_in_body,
        grid=(3, GRID),
        in_specs=[
            pl.BlockSpec((BR, 32), row), pl.BlockSpec((BR, 16), row), pl.BlockSpec((BR, 8), row),
            pl.BlockSpec((32, HID), whole), pl.BlockSpec((1, HID), whole),
            pl.BlockSpec((16, HID), whole), pl.BlockSpec((1, HID), whole),
            pl.BlockSpec((8, HID), whole), pl.BlockSpec((1, HID), whole),
        ],
        out_specs=pl.BlockSpec((BR, HID), lambda t, i: (t * GRID + i, 0)),
        out_shape=jax.ShapeDtypeStruct((3 * N, HID), jnp.float32),
    )(xt, xv, xr, wt, bt, wv, bv, wr, br)


def _proj_cat_body(x, w, b, o):
    o[...] = jnp.dot(x[...], w[0], preferred_element_type=jnp.float32) + b[0]


def _proj_cat(x_cat, wcat, bcat):
    return pl.pallas_call(
        _proj_cat_body,
        grid=(3, GRID),
        in_specs=[
            pl.BlockSpec((BR, HID), lambda t, i: (t * GRID + i, 0)),
            pl.BlockSpec((1, HID, 6 * HID), lambda t, i: (t, 0, 0)),
            pl.BlockSpec((1, 1, 6 * HID), lambda t, i: (t, 0, 0)),
        ],
        out_specs=pl.BlockSpec((BR, 6 * HID), lambda t, i: (t * GRID + i, 0)),
        out_shape=jax.ShapeDtypeStruct((3 * N, 6 * HID), jnp.float32),
    )(x_cat, wcat, bcat)


def _merge_body(p00, p01, p02, p10, p11, p12, b0, b1, b2, out, *, relu):
    acc = None
    for (a, b, bias) in ((p00, p10, b0), (p01, p11, b1), (p02, p12, b2)):
        q = a[0, 0] + b[0, 0]
        s0 = q[:, HID:HID + 1] + 1e-30
        s1 = q[:, HID + 1:HID + 2] + 1e-30
        o = jnp.concatenate([q[:, :CH] / s0, q[:, CH:HID] / s1], axis=-1)
        o = o + bias[0]
        acc = o if acc is None else acc + o
    out[...] = jnp.maximum(acc, 0.0) if relu else acc


def _merge(partials, bias_all, relu):
    def rel_of(t, slot):
        v = DST_REL_IDS
        return jnp.where(t == 0, v[0][slot], jnp.where(t == 1, v[1][slot], v[2][slot]))

    specs = []
    for sc in (0, 1):
        for slot in range(3):
            specs.append(pl.BlockSpec(
                (1, 1, BR, WACC),
                functools.partial(lambda t, i, _sc, _k: (_sc, rel_of(t, _k), i, 0),
                                  _sc=sc, _k=slot)))
    for slot in range(3):
        specs.append(pl.BlockSpec(
            (1, 1, HID),
            functools.partial(lambda t, i, _k: (rel_of(t, _k), 0, 0), _k=slot)))
    return pl.pallas_call(
        functools.partial(_merge_body, relu=relu),
        grid=(3, GRID),
        in_specs=specs,
        out_specs=pl.BlockSpec((BR, HID), lambda t, i: (t * GRID + i, 0)),
        out_shape=jax.ShapeDtypeStruct((3 * N, HID), jnp.float32),
    )(partials, partials, partials, partials, partials, partials,
      bias_all, bias_all, bias_all)


def _tail_body(xcat, wsc, bsc, wro, bro, wrb, brb, w1, b1, w2, b2,
               svc, road, rob, g, glob):
    xt = xcat[0:N]
    xv = xcat[N:2 * N]
    xr = xcat[2 * N:3 * N]
    svc[...] = jnp.dot(xv, wsc[...], preferred_element_type=jnp.float32) + bsc[...]
    road[...] = jnp.dot(xr, wro[...], preferred_element_type=jnp.float32) + bro[...]
    rob[...] = jnp.dot(xt, wrb[...], preferred_element_type=jnp.float32) + brb[...]
    mt = jnp.mean(xt, axis=0, keepdims=True)
    mv = jnp.mean(xv, axis=0, keepdims=True)
    mr = jnp.mean(xr, axis=0, keepdims=True)
    gg = jnp.concatenate([mt, mv, mr], axis=-1)
    g[...] = gg
    h = jnp.maximum(jnp.dot(gg, w1[...], preferred_element_type=jnp.float32) + b1[...], 0.0)
    glob[...] = jnp.dot(h, w2[...], preferred_element_type=jnp.float32) + b2[...]


def _tail(xcat, p):
    wsc = jnp.concatenate([p['head_settlement']['W'], p['head_city']['W']], axis=1)
    bsc = jnp.stack([p['head_settlement']['b'][0], p['head_city']['b'][0]]).reshape(1, 2)
    ins = (xcat, wsc, bsc,
           p['head_road']['W'], p['head_road']['b'].reshape(1, 1),
           p['head_robber']['W'], p['head_robber']['b'].reshape(1, 1),
           p['glob1']['W'], p['glob1']['b'].reshape(1, HID),
           p['glob2']['W'], p['glob2']['b'].reshape(1, 2))
    return pl.pallas_call(
        _tail_body,
        out_shape=[
            jax.ShapeDtypeStruct((N, 2), jnp.float32),
            jax.ShapeDtypeStruct((N, 1), jnp.float32),
            jax.ShapeDtypeStruct((N, 1), jnp.float32),
            jax.ShapeDtypeStruct((1, 3 * HID), jnp.float32),
            jax.ShapeDtypeStruct((1, 2), jnp.float32),
        ],
    )(*ins)


def _sc_edge_body(tab, sidx, didx, scat, att, zrows, out,
                  acc, rows_l, rows_r, wrow, zbuf, att_vb,
                  sidx_v, didx_v, scat_v, sem1, sem2):
    cid = lax.axis_index("c")
    sid = lax.axis_index("s")
    wid = cid * 16 + sid
    io16 = jnp.arange(16, dtype=jnp.int32)

    pltpu.sync_copy(zrows, zbuf)

    def rel_body(rel, carry):
        def zero_body(z, c):
            pltpu.sync_copy(zbuf, acc.at[pl.ds(sid * 640 + z * ZROWS, ZROWS)])
            return c
        lax.fori_loop(0, NZCH, zero_body, 0)
        plsc.subcore_barrier()

        pltpu.sync_copy(att.at[pl.ds(rel * HID, HID)], att_vb)
        att_vecs = [att_vb[pl.ds(16 * j, 16)] for j in range(8)]

        def edge_body(e, c):
            ls = [rows_l[e, pl.ds(16 * j, 16)] for j in range(8)]
            rs = [rows_r[e, pl.ds(16 * j, 16)] for j in range(8)]
            ps = []
            for j in range(8):
                t = ls[j] + rs[j]
                t = jnp.maximum(t, 0.2 * t)
                ps.append(t * att_vecs[j])
            ha = (ps[0] + ps[1]) + (ps[2] + ps[3])
            hb = (ps[4] + ps[5]) + (ps[6] + ps[7])
            ea = jnp.exp(jnp.full((16,), jnp.sum(ha), jnp.float32))
            eb = jnp.exp(jnp.full((16,), jnp.sum(hb), jnp.float32))
            for j in range(4):
                wrow[e, pl.ds(16 * j, 16)] = ls[j] * ea
            for j in range(4, 8):
                wrow[e, pl.ds(16 * j, 16)] = ls[j] * eb
            wrow[e, pl.ds(HID, 16)] = (jnp.where(io16 == 0, ea, 0.0)
                                       + jnp.where(io16 == 1, eb, 0.0))
            return c

        def chunk_body(ch, c):
            off = rel * EPAD + wid * EPW + ch * BLK
            pltpu.sync_copy(sidx.at[pl.ds(off, BLK)], sidx_v)
            pltpu.sync_copy(didx.at[pl.ds(off, BLK)], didx_v)
            pltpu.sync_copy(scat.at[pl.ds(off, BLK)], scat_v)
            c1 = pltpu.async_copy(tab.at[sidx_v], rows_l, sem1)
            c2 = pltpu.async_copy(tab.at[didx_v], rows_r, sem2)
            c1.wait()
            c2.wait()
            lax.fori_loop(0, BLK, edge_body, 0)
            pltpu.sync_copy(wrow, acc.at[scat_v], add=True)
            return c

        lax.fori_loop(0, NCHUNK, chunk_body, 0)
        plsc.subcore_barrier()

        pltpu.sync_copy(acc.at[pl.ds(sid * 640, 640)],
                        out.at[cid, rel, pl.ds(sid * 640, 640), :])
        plsc.subcore_barrier()
        return carry

    lax.fori_loop(0, 9, rel_body, 0)


def _sc_edge_layer(tab, sidx, didx, scat, att, zrows):
    mesh = plsc.VectorSubcoreMesh(core_axis_name="c", subcore_axis_name="s",
                                  num_cores=2, num_subcores=16)
    return pl.kernel(
        _sc_edge_body,
        out_type=jax.ShapeDtypeStruct((2, 9, ACC_ROWS, WACC), jnp.float32),
        mesh=mesh,
        compiler_params=pltpu.CompilerParams(use_tc_tiling_on_sc=False,
                                             needs_layout_passes=False),
        scratch_types=[
            pltpu.VMEM_SHARED((ACC_ROWS, WACC), jnp.float32),
            pltpu.VMEM((BLK, HID), jnp.float32),
            pltpu.VMEM((BLK, HID), jnp.float32),
            pltpu.VMEM((BLK, WACC), jnp.float32),
            pltpu.VMEM((ZROWS, WACC), jnp.float32),
            pltpu.VMEM((HID,), jnp.float32),
            pltpu.VMEM((BLK,), jnp.int32),
            pltpu.VMEM((BLK,), jnp.int32),
            pltpu.VMEM((BLK,), jnp.int32),
            pltpu.SemaphoreType.DMA,
            pltpu.SemaphoreType.DMA,
        ],
    )(tab, sidx, didx, scat, att, zrows)


def _edge_indices(eis):
    loops = jnp.arange(N, dtype=jnp.int32)
    padz = jnp.zeros((EPAD - ETOT,), jnp.int32)
    padg = jnp.full((EPAD - ETOT,), GARBAGE, jnp.int32)
    sidx, didx, scat = [], [], []
    for r in RELS:
        ei = eis[r]
        src = jnp.concatenate([ei[0].astype(jnp.int32), loops, padz])
        dst = jnp.concatenate([ei[1].astype(jnp.int32), loops])
        sidx.append(TIDX[r[0]] * 6 * N + src * 6 + L_SLOT[r])
        didx.append(TIDX[r[1]] * 6 * N + jnp.concatenate([dst, padz]) * 6 + R_SLOT[r])
        scat.append(jnp.concatenate([dst, padg]))
    return (jnp.concatenate(sidx), jnp.concatenate(didx), jnp.concatenate(scat))


def _layer_weights(lp):
    wcat, bcat = [], []
    for t in TYPES:
        wcat.append(jnp.concatenate(
            [lp[r]['Wl'] for r in SRC_RELS[t]] + [lp[r]['Wr'] for r in DST_RELS[t]],
            axis=1))
        bcat.append(jnp.concatenate(
            [lp[r]['bl'] for r in SRC_RELS[t]] + [lp[r]['br'] for r in DST_RELS[t]]))
    return jnp.stack(wcat), jnp.stack(bcat).reshape(3, 1, 6 * HID)


def _gnn_layer(x_cat, lp, idxs, zrows, relu):
    wcat, bcat = _layer_weights(lp)
    y = _proj_cat(x_cat, wcat, bcat)
    tab = y.reshape(18 * N, HID)
    att = jnp.concatenate([lp[r]['att'].reshape(HID) for r in RELS])
    partials = _sc_edge_layer(tab, idxs[0], idxs[1], idxs[2], att, zrows)
    bias_all = jnp.stack([lp[r]['bias'] for r in RELS]).reshape(9, 1, HID)
    return _merge(partials, bias_all, relu)


def kernel(x_tile, x_vertex, x_road, ei_tt, ei_tv, ei_vt, ei_tr, ei_rt,
           ei_vr, ei_rv, ei_vv, ei_rr, params: Any):
    eis = {'tt': ei_tt, 'tv': ei_tv, 'vt': ei_vt, 'tr': ei_tr, 'rt': ei_rt,
           'vr': ei_vr, 'rv': ei_rv, 'vv': ei_vv, 'rr': ei_rr}
    idxs = _edge_indices(eis)
    zrows = jnp.zeros((ZROWS, WACC), jnp.float32)

    x_cat = _proj_in(x_tile, x_vertex, x_road, params)
    x_cat = _gnn_layer(x_cat, params['gnn1'], idxs, zrows, relu=True)
    x_cat = _gnn_layer(x_cat, params['gnn2'], idxs, zrows, relu=False)

    svc, road, rob, g, glob = _tail(x_cat, params)
    return (svc[:, 0], svc[:, 1], road[:, 0], rob[:, 0], glob[0],
            x_cat[0:N], x_cat[N:2 * N], x_cat[2 * N:3 * N], g)


# speedup vs baseline: 25.8302x; 1.3512x over previous
import functools
from typing import Any

import jax
import jax.numpy as jnp
from jax import lax
from jax.experimental import pallas as pl
from jax.experimental.pallas import tpu as pltpu
from jax.experimental.pallas import tpu_sc as plsc

N = 10000
E = 64000
ETOT = E + N
HID = 128
NHEAD = 2
CH = 64
RELS = ('tt', 'tv', 'vt', 'tr', 'rt', 'vr', 'rv', 'vv', 'rr')
TYPES = ('t', 'v', 'r')

NWORK = 32
BLK = 48
NCHUNK = 50
NPAIR = NCHUNK // 2
EPW = BLK * NCHUNK
EPAD = EPW * NWORK
ACC_ROWS = 10240
WACC = 144
GARBAGE = 10008
ZROWS = 32
NZCH = 20

SRC_RELS = {t: [r for r in RELS if r[0] == t] for t in TYPES}
DST_RELS = {t: [r for r in RELS if r[1] == t] for t in TYPES}
L_SLOT = {r: SRC_RELS[r[0]].index(r) for r in RELS}
R_SLOT = {r: 3 + DST_RELS[r[1]].index(r) for r in RELS}
TIDX = {'t': 0, 'v': 1, 'r': 2}
DST_REL_IDS = [[RELS.index(r) for r in DST_RELS[t]] for t in TYPES]

BR = 1000
GRID = N // BR


def _proj_in_body(xt, xv, xr, wt, bt, wv, bv, wr, br, out):
    t = pl.program_id(0)

    def mk(x, w, b):
        def f():
            out[...] = jnp.maximum(
                jnp.dot(x[...], w[...], preferred_element_type=jnp.float32) + b[...], 0.0)
        return f

    branches = [mk(xt, wt, bt), mk(xv, wv, bv), mk(xr, wr, br)]
    lax.switch(t, branches)


def _proj_in(xt, xv, xr, p):
    def w(name):
        return p[name]['W'], p[name]['b'].reshape(1, HID)
    wt, bt = w('in_tile')
    wv, bv = w('in_vertex')
    wr, br = w('in_road')
    row = lambda t, i: (i, 0)
    whole = lambda t, i: (0, 0)
    return pl.pallas_call(
        _proj_in_body,
        grid=(3, GRID),
        in_specs=[
            pl.BlockSpec((BR, 32), row), pl.BlockSpec((BR, 16), row), pl.BlockSpec((BR, 8), row),
            pl.BlockSpec((32, HID), whole), pl.BlockSpec((1, HID), whole),
            pl.BlockSpec((16, HID), whole), pl.BlockSpec((1, HID), whole),
            pl.BlockSpec((8, HID), whole), pl.BlockSpec((1, HID), whole),
        ],
        out_specs=pl.BlockSpec((BR, HID), lambda t, i: (t * GRID + i, 0)),
        out_shape=jax.ShapeDtypeStruct((3 * N, HID), jnp.float32),
    )(xt, xv, xr, wt, bt, wv, bv, wr, br)


def _proj_cat_body(x, w, b, o):
    o[...] = jnp.dot(x[...], w[0], preferred_element_type=jnp.float32) + b[0]


def _proj_cat(x_cat, wcat, bcat):
    return pl.pallas_call(
        _proj_cat_body,
        grid=(3, GRID),
        in_specs=[
            pl.BlockSpec((BR, HID), lambda t, i: (t * GRID + i, 0)),
            pl.BlockSpec((1, HID, 6 * HID), lambda t, i: (t, 0, 0)),
            pl.BlockSpec((1, 1, 6 * HID), lambda t, i: (t, 0, 0)),
        ],
        out_specs=pl.BlockSpec((BR, 6 * HID), lambda t, i: (t * GRID + i, 0)),
        out_shape=jax.ShapeDtypeStruct((3 * N, 6 * HID), jnp.float32),
    )(x_cat, wcat, bcat)


def _merge_body(p00, p01, p02, p10, p11, p12, b0, b1, b2, out, *, relu):
    acc = None
    for (a, b, bias) in ((p00, p10, b0), (p01, p11, b1), (p02, p12, b2)):
        q = a[0, 0] + b[0, 0]
        s0 = q[:, HID:HID + 1] + 1e-30
        s1 = q[:, HID + 1:HID + 2] + 1e-30
        o = jnp.concatenate([q[:, :CH] / s0, q[:, CH:HID] / s1], axis=-1)
        o = o + bias[0]
        acc = o if acc is None else acc + o
    out[...] = jnp.maximum(acc, 0.0) if relu else acc


def _merge(partials, bias_all, relu):
    def rel_of(t, slot):
        v = DST_REL_IDS
        return jnp.where(t == 0, v[0][slot], jnp.where(t == 1, v[1][slot], v[2][slot]))

    specs = []
    for sc in (0, 1):
        for slot in range(3):
            specs.append(pl.BlockSpec(
                (1, 1, BR, WACC),
                functools.partial(lambda t, i, _sc, _k: (_sc, rel_of(t, _k), i, 0),
                                  _sc=sc, _k=slot)))
    for slot in range(3):
        specs.append(pl.BlockSpec(
            (1, 1, HID),
            functools.partial(lambda t, i, _k: (rel_of(t, _k), 0, 0), _k=slot)))
    return pl.pallas_call(
        functools.partial(_merge_body, relu=relu),
        grid=(3, GRID),
        in_specs=specs,
        out_specs=pl.BlockSpec((BR, HID), lambda t, i: (t * GRID + i, 0)),
        out_shape=jax.ShapeDtypeStruct((3 * N, HID), jnp.float32),
    )(partials, partials, partials, partials, partials, partials,
      bias_all, bias_all, bias_all)


def _tail_body(xcat, wsc, bsc, wro, bro, wrb, brb, w1, b1, w2, b2,
               svc, road, rob, g, glob):
    xt = xcat[0:N]
    xv = xcat[N:2 * N]
    xr = xcat[2 * N:3 * N]
    svc[...] = jnp.dot(xv, wsc[...], preferred_element_type=jnp.float32) + bsc[...]
    road[...] = jnp.dot(xr, wro[...], preferred_element_type=jnp.float32) + bro[...]
    rob[...] = jnp.dot(xt, wrb[...], preferred_element_type=jnp.float32) + brb[...]
    mt = jnp.mean(xt, axis=0, keepdims=True)
    mv = jnp.mean(xv, axis=0, keepdims=True)
    mr = jnp.mean(xr, axis=0, keepdims=True)
    gg = jnp.concatenate([mt, mv, mr], axis=-1)
    g[...] = gg
    h = jnp.maximum(jnp.dot(gg, w1[...], preferred_element_type=jnp.float32) + b1[...], 0.0)
    glob[...] = jnp.dot(h, w2[...], preferred_element_type=jnp.float32) + b2[...]


def _tail(xcat, p):
    wsc = jnp.concatenate([p['head_settlement']['W'], p['head_city']['W']], axis=1)
    bsc = jnp.stack([p['head_settlement']['b'][0], p['head_city']['b'][0]]).reshape(1, 2)
    ins = (xcat, wsc, bsc,
           p['head_road']['W'], p['head_road']['b'].reshape(1, 1),
           p['head_robber']['W'], p['head_robber']['b'].reshape(1, 1),
           p['glob1']['W'], p['glob1']['b'].reshape(1, HID),
           p['glob2']['W'], p['glob2']['b'].reshape(1, 2))
    return pl.pallas_call(
        _tail_body,
        out_shape=[
            jax.ShapeDtypeStruct((N, 2), jnp.float32),
            jax.ShapeDtypeStruct((N, 1), jnp.float32),
            jax.ShapeDtypeStruct((N, 1), jnp.float32),
            jax.ShapeDtypeStruct((1, 3 * HID), jnp.float32),
            jax.ShapeDtypeStruct((1, 2), jnp.float32),
        ],
    )(*ins)


def _sc_edge_body(tab, idxall, att, zrows, out,
                  acc, rows_la, rows_ra, rows_lb, rows_rb, wrow, zbuf, att_vb,
                  idx_a, idx_b, sem_la, sem_ra, sem_lb, sem_rb, zsem):
    cid = lax.axis_index("c")
    sid = lax.axis_index("s")
    wid = cid * 16 + sid
    io16 = jnp.arange(16, dtype=jnp.int32)

    pltpu.sync_copy(zrows, zbuf)

    def rel_body(rel, carry):
        zds = [pltpu.async_copy(zbuf, acc.at[pl.ds(sid * 640 + z * ZROWS, ZROWS)],
                                zsem) for z in range(NZCH)]
        for zd in zds:
            zd.wait()
        plsc.subcore_barrier()

        pltpu.sync_copy(att.at[pl.ds(rel * HID, HID)], att_vb)
        att_vecs = [att_vb[pl.ds(16 * j, 16)] for j in range(8)]
        chunk0 = (rel * NWORK + wid) * NCHUNK

        def compute_scatter(rows_l, rows_r, idx_v):
            def edge_body(e, c):
                ls = [rows_l[e, pl.ds(16 * j, 16)] for j in range(8)]
                rs = [rows_r[e, pl.ds(16 * j, 16)] for j in range(8)]
                ps = []
                for j in range(8):
                    t = ls[j] + rs[j]
                    t = jnp.maximum(t, 0.2 * t)
                    ps.append(t * att_vecs[j])
                ha = (ps[0] + ps[1]) + (ps[2] + ps[3])
                hb = (ps[4] + ps[5]) + (ps[6] + ps[7])
                ea = jnp.exp(jnp.full((16,), jnp.sum(ha), jnp.float32))
                eb = jnp.exp(jnp.full((16,), jnp.sum(hb), jnp.float32))
                for j in range(4):
                    wrow[e, pl.ds(16 * j, 16)] = ls[j] * ea
                for j in range(4, 8):
                    wrow[e, pl.ds(16 * j, 16)] = ls[j] * eb
                wrow[e, pl.ds(HID, 16)] = (jnp.where(io16 == 0, ea, 0.0)
                                           + jnp.where(io16 == 1, eb, 0.0))
                return c
            lax.fori_loop(0, BLK, edge_body, 0)
            pltpu.sync_copy(wrow, acc.at[idx_v.at[2]], add=True)

        def fetch(ch, idx_v, rows_l, rows_r, sl, sr):
            pltpu.sync_copy(idxall.at[chunk0 + ch], idx_v)
            pltpu.async_copy(tab.at[idx_v.at[0]], rows_l, sl)
            pltpu.async_copy(tab.at[idx_v.at[1]], rows_r, sr)

        def wait_ab(idx_v, rows_l, rows_r, sl, sr):
            pltpu.make_async_copy(tab.at[idx_v.at[0]], rows_l, sl).wait()
            pltpu.make_async_copy(tab.at[idx_v.at[1]], rows_r, sr).wait()

        fetch(0, idx_a, rows_la, rows_ra, sem_la, sem_ra)

        def pair_body(p, c):
            fetch(2 * p + 1, idx_b, rows_lb, rows_rb, sem_lb, sem_rb)
            wait_ab(idx_a, rows_la, rows_ra, sem_la, sem_ra)
            compute_scatter(rows_la, rows_ra, idx_a)

            @pl.when(p < NPAIR - 1)
            def _prefetch_a():
                fetch(2 * p + 2, idx_a, rows_la, rows_ra, sem_la, sem_ra)

            wait_ab(idx_b, rows_lb, rows_rb, sem_lb, sem_rb)
            compute_scatter(rows_lb, rows_rb, idx_b)
            return c

        lax.fori_loop(0, NPAIR, pair_body, 0)
        plsc.subcore_barrier()

        pltpu.sync_copy(acc.at[pl.ds(sid * 640, 640)],
                        out.at[cid, rel, pl.ds(sid * 640, 640), :])
        plsc.subcore_barrier()
        return carry

    lax.fori_loop(0, 9, rel_body, 0)


def _sc_edge_layer(tab, idxall, att, zrows):
    mesh = plsc.VectorSubcoreMesh(core_axis_name="c", subcore_axis_name="s",
                                  num_cores=2, num_subcores=16)
    return pl.kernel(
        _sc_edge_body,
        out_type=jax.ShapeDtypeStruct((2, 9, ACC_ROWS, WACC), jnp.float32),
        mesh=mesh,
        compiler_params=pltpu.CompilerParams(use_tc_tiling_on_sc=False,
                                             needs_layout_passes=False),
        scratch_types=[
            pltpu.VMEM_SHARED((ACC_ROWS, WACC), jnp.float32),
            pltpu.VMEM((BLK, HID), jnp.float32),
            pltpu.VMEM((BLK, HID), jnp.float32),
            pltpu.VMEM((BLK, HID), jnp.float32),
            pltpu.VMEM((BLK, HID), jnp.float32),
            pltpu.VMEM((BLK, WACC), jnp.float32),
            pltpu.VMEM((ZROWS, WACC), jnp.float32),
            pltpu.VMEM((HID,), jnp.float32),
            pltpu.VMEM((4, BLK), jnp.int32),
            pltpu.VMEM((4, BLK), jnp.int32),
            pltpu.SemaphoreType.DMA,
            pltpu.SemaphoreType.DMA,
            pltpu.SemaphoreType.DMA,
            pltpu.SemaphoreType.DMA,
            pltpu.SemaphoreType.DMA,
        ],
    )(tab, idxall, att, zrows)


def _edge_indices(eis):
    loops = jnp.arange(N, dtype=jnp.int32)
    padz = jnp.zeros((EPAD - ETOT,), jnp.int32)
    padg = jnp.full((EPAD - ETOT,), GARBAGE, jnp.int32)
    zline = jnp.zeros((EPAD,), jnp.int32)
    rows = []
    for r in RELS:
        ei = eis[r]
        src = jnp.concatenate([ei[0].astype(jnp.int32), loops, padz])
        dst = jnp.concatenate([ei[1].astype(jnp.int32), loops])
        sidx = TIDX[r[0]] * 6 * N + src * 6 + L_SLOT[r]
        didx = TIDX[r[1]] * 6 * N + jnp.concatenate([dst, padz]) * 6 + R_SLOT[r]
        scat = jnp.concatenate([dst, padg])
        quad = jnp.stack([sidx, didx, scat, zline])
        rows.append(quad.reshape(4, NWORK, NCHUNK, BLK).transpose(1, 2, 0, 3))
    return jnp.stack(rows).reshape(9 * NWORK * NCHUNK, 4, BLK)


def _layer_weights(lp):
    wcat, bcat = [], []
    for t in TYPES:
        wcat.append(jnp.concatenate(
            [lp[r]['Wl'] for r in SRC_RELS[t]] + [lp[r]['Wr'] for r in DST_RELS[t]],
            axis=1))
        bcat.append(jnp.concatenate(
            [lp[r]['bl'] for r in SRC_RELS[t]] + [lp[r]['br'] for r in DST_RELS[t]]))
    return jnp.stack(wcat), jnp.stack(bcat).reshape(3, 1, 6 * HID)


def _gnn_layer(x_cat, lp, idxs, zrows, relu):
    wcat, bcat = _layer_weights(lp)
    y = _proj_cat(x_cat, wcat, bcat)
    tab = y.reshape(18 * N, HID)
    att = jnp.concatenate([lp[r]['att'].reshape(HID) for r in RELS])
    partials = _sc_edge_layer(tab, idxs, att, zrows)
    bias_all = jnp.stack([lp[r]['bias'] for r in RELS]).reshape(9, 1, HID)
    return _merge(partials, bias_all, relu)


def kernel(x_tile, x_vertex, x_road, ei_tt, ei_tv, ei_vt, ei_tr, ei_rt,
           ei_vr, ei_rv, ei_vv, ei_rr, params: Any):
    eis = {'tt': ei_tt, 'tv': ei_tv, 'vt': ei_vt, 'tr': ei_tr, 'rt': ei_rt,
           'vr': ei_vr, 'rv': ei_rv, 'vv': ei_vv, 'rr': ei_rr}
    idxs = _edge_indices(eis)
    zrows = jnp.zeros((ZROWS, WACC), jnp.float32)

    x_cat = _proj_in(x_tile, x_vertex, x_road, params)
    x_cat = _gnn_layer(x_cat, params['gnn1'], idxs, zrows, relu=True)
    x_cat = _gnn_layer(x_cat, params['gnn2'], idxs, zrows, relu=False)

    svc, road, rob, g, glob = _tail(x_cat, params)
    return (svc[:, 0], svc[:, 1], road[:, 0], rob[:, 0], glob[0],
            x_cat[0:N], x_cat[N:2 * N], x_cat[2 * N:3 * N], g)


# speedup vs baseline: 26.2034x; 1.0144x over previous
import functools
from typing import Any

import jax
import jax.numpy as jnp
from jax import lax
from jax.experimental import pallas as pl
from jax.experimental.pallas import tpu as pltpu
from jax.experimental.pallas import tpu_sc as plsc

N = 10000
E = 64000
ETOT = E + N
HID = 128
NHEAD = 2
CH = 64
RELS = ('tt', 'tv', 'vt', 'tr', 'rt', 'vr', 'rv', 'vv', 'rr')
TYPES = ('t', 'v', 'r')

NWORK = 32
BLK = 48
NCHUNK = 50
NPAIR = NCHUNK // 2
EPW = BLK * NCHUNK
EPAD = EPW * NWORK
ACC_ROWS = 10240
WACC = 144
GARBAGE = 10008
ZROWS = 32
NZCH = 20

SRC_RELS = {t: [r for r in RELS if r[0] == t] for t in TYPES}
DST_RELS = {t: [r for r in RELS if r[1] == t] for t in TYPES}
L_SLOT = {r: SRC_RELS[r[0]].index(r) for r in RELS}
R_SLOT = {r: 3 + DST_RELS[r[1]].index(r) for r in RELS}
TIDX = {'t': 0, 'v': 1, 'r': 2}
DST_REL_IDS = [[RELS.index(r) for r in DST_RELS[t]] for t in TYPES]

BR = 1000
GRID = N // BR


def _proj_in_body(xt, xv, xr, wt, bt, wv, bv, wr, br, out):
    t = pl.program_id(0)

    def mk(x, w, b):
        def f():
            out[...] = jnp.maximum(
                jnp.dot(x[...], w[...], preferred_element_type=jnp.float32) + b[...], 0.0)
        return f

    branches = [mk(xt, wt, bt), mk(xv, wv, bv), mk(xr, wr, br)]
    lax.switch(t, branches)


def _proj_in(xt, xv, xr, p):
    def w(name):
        return p[name]['W'], p[name]['b'].reshape(1, HID)
    wt, bt = w('in_tile')
    wv, bv = w('in_vertex')
    wr, br = w('in_road')
    row = lambda t, i: (i, 0)
    whole = lambda t, i: (0, 0)
    return pl.pallas_call(
        _proj_in_body,
        grid=(3, GRID),
        in_specs=[
            pl.BlockSpec((BR, 32), row), pl.BlockSpec((BR, 16), row), pl.BlockSpec((BR, 8), row),
            pl.BlockSpec((32, HID), whole), pl.BlockSpec((1, HID), whole),
            pl.BlockSpec((16, HID), whole), pl.BlockSpec((1, HID), whole),
            pl.BlockSpec((8, HID), whole), pl.BlockSpec((1, HID), whole),
        ],
        out_specs=pl.BlockSpec((BR, HID), lambda t, i: (t * GRID + i, 0)),
        out_shape=jax.ShapeDtypeStruct((3 * N, HID), jnp.float32),
    )(xt, xv, xr, wt, bt, wv, bv, wr, br)


def _proj_cat_body(x, w, b, o):
    o[...] = jnp.dot(x[...], w[0], preferred_element_type=jnp.float32) + b[0]


def _proj_cat(x_cat, wcat, bcat):
    return pl.pallas_call(
        _proj_cat_body,
        grid=(3, GRID),
        in_specs=[
            pl.BlockSpec((BR, HID), lambda t, i: (t * GRID + i, 0)),
            pl.BlockSpec((1, HID, 6 * HID), lambda t, i: (t, 0, 0)),
            pl.BlockSpec((1, 1, 6 * HID), lambda t, i: (t, 0, 0)),
        ],
        out_specs=pl.BlockSpec((BR, 6 * HID), lambda t, i: (t * GRID + i, 0)),
        out_shape=jax.ShapeDtypeStruct((3 * N, 6 * HID), jnp.float32),
    )(x_cat, wcat, bcat)


def _merge_body(p00, p01, p02, p10, p11, p12, b0, b1, b2, out, *, relu):
    acc = None
    for (a, b, bias) in ((p00, p10, b0), (p01, p11, b1), (p02, p12, b2)):
        q = a[0, 0] + b[0, 0]
        s0 = q[:, HID:HID + 1] + 1e-30
        s1 = q[:, HID + 1:HID + 2] + 1e-30
        o = jnp.concatenate([q[:, :CH] / s0, q[:, CH:HID] / s1], axis=-1)
        o = o + bias[0]
        acc = o if acc is None else acc + o
    out[...] = jnp.maximum(acc, 0.0) if relu else acc


def _merge(partials, bias_all, relu):
    def rel_of(t, slot):
        v = DST_REL_IDS
        return jnp.where(t == 0, v[0][slot], jnp.where(t == 1, v[1][slot], v[2][slot]))

    specs = []
    for sc in (0, 1):
        for slot in range(3):
            specs.append(pl.BlockSpec(
                (1, 1, BR, WACC),
                functools.partial(lambda t, i, _sc, _k: (_sc, rel_of(t, _k), i, 0),
                                  _sc=sc, _k=slot)))
    for slot in range(3):
        specs.append(pl.BlockSpec(
            (1, 1, HID),
            functools.partial(lambda t, i, _k: (rel_of(t, _k), 0, 0), _k=slot)))
    return pl.pallas_call(
        functools.partial(_merge_body, relu=relu),
        grid=(3, GRID),
        in_specs=specs,
        out_specs=pl.BlockSpec((BR, HID), lambda t, i: (t * GRID + i, 0)),
        out_shape=jax.ShapeDtypeStruct((3 * N, HID), jnp.float32),
    )(partials, partials, partials, partials, partials, partials,
      bias_all, bias_all, bias_all)


def _tail_body(xcat, wsc, bsc, wro, bro, wrb, brb, w1, b1, w2, b2,
               svc, road, rob, g, glob):
    xt = xcat[0:N]
    xv = xcat[N:2 * N]
    xr = xcat[2 * N:3 * N]
    svc[...] = jnp.dot(xv, wsc[...], preferred_element_type=jnp.float32) + bsc[...]
    road[...] = jnp.dot(xr, wro[...], preferred_element_type=jnp.float32) + bro[...]
    rob[...] = jnp.dot(xt, wrb[...], preferred_element_type=jnp.float32) + brb[...]
    mt = jnp.mean(xt, axis=0, keepdims=True)
    mv = jnp.mean(xv, axis=0, keepdims=True)
    mr = jnp.mean(xr, axis=0, keepdims=True)
    gg = jnp.concatenate([mt, mv, mr], axis=-1)
    g[...] = gg
    h = jnp.maximum(jnp.dot(gg, w1[...], preferred_element_type=jnp.float32) + b1[...], 0.0)
    glob[...] = jnp.dot(h, w2[...], preferred_element_type=jnp.float32) + b2[...]


def _tail(xcat, p):
    wsc = jnp.concatenate([p['head_settlement']['W'], p['head_city']['W']], axis=1)
    bsc = jnp.stack([p['head_settlement']['b'][0], p['head_city']['b'][0]]).reshape(1, 2)
    ins = (xcat, wsc, bsc,
           p['head_road']['W'], p['head_road']['b'].reshape(1, 1),
           p['head_robber']['W'], p['head_robber']['b'].reshape(1, 1),
           p['glob1']['W'], p['glob1']['b'].reshape(1, HID),
           p['glob2']['W'], p['glob2']['b'].reshape(1, 2))
    return pl.pallas_call(
        _tail_body,
        out_shape=[
            jax.ShapeDtypeStruct((N, 2), jnp.float32),
            jax.ShapeDtypeStruct((N, 1), jnp.float32),
            jax.ShapeDtypeStruct((N, 1), jnp.float32),
            jax.ShapeDtypeStruct((1, 3 * HID), jnp.float32),
            jax.ShapeDtypeStruct((1, 2), jnp.float32),
        ],
    )(*ins)


def _sc_edge_body(tab, idxall, att, zrows, out,
                  acc, rows_la, rows_ra, rows_lb, rows_rb, wrow, zbuf, att_vb,
                  idx_a, idx_b, scat_a, scat_b,
                  sem_la, sem_ra, sem_lb, sem_rb, zsem, ssem):
    cid = lax.axis_index("c")
    sid = lax.axis_index("s")
    wid = cid * 16 + sid
    io16 = jnp.arange(16, dtype=jnp.int32)

    pltpu.sync_copy(zrows, zbuf)

    def rel_body(rel, carry):
        zds = [pltpu.async_copy(zbuf, acc.at[pl.ds(sid * 640 + z * ZROWS, ZROWS)],
                                zsem) for z in range(NZCH)]
        for zd in zds:
            zd.wait()
        plsc.subcore_barrier()

        pltpu.sync_copy(att.at[pl.ds(rel * HID, HID)], att_vb)
        att_vecs = [att_vb[pl.ds(16 * j, 16)] for j in range(8)]
        chunk0 = (rel * NWORK + wid) * NCHUNK

        def compute(rows_l, rows_r, idx_v):
            def edge_body(e, c):
                ls = [rows_l[e, pl.ds(16 * j, 16)] for j in range(8)]
                rs = [rows_r[e, pl.ds(16 * j, 16)] for j in range(8)]
                ps = []
                for j in range(8):
                    t = ls[j] + rs[j]
                    t = jnp.maximum(t, 0.2 * t)
                    ps.append(t * att_vecs[j])
                ha = (ps[0] + ps[1]) + (ps[2] + ps[3])
                hb = (ps[4] + ps[5]) + (ps[6] + ps[7])
                ea = jnp.exp(jnp.full((16,), jnp.sum(ha), jnp.float32))
                eb = jnp.exp(jnp.full((16,), jnp.sum(hb), jnp.float32))
                for j in range(4):
                    wrow[e, pl.ds(16 * j, 16)] = ls[j] * ea
                for j in range(4, 8):
                    wrow[e, pl.ds(16 * j, 16)] = ls[j] * eb
                wrow[e, pl.ds(HID, 16)] = (jnp.where(io16 == 0, ea, 0.0)
                                           + jnp.where(io16 == 1, eb, 0.0))
                return c
            lax.fori_loop(0, BLK, edge_body, 0)

        def fetch(ch, idx_v, rows_l, rows_r, sl, sr):
            pltpu.sync_copy(idxall.at[chunk0 + ch], idx_v)
            pltpu.async_copy(tab.at[idx_v.at[0]], rows_l, sl)
            pltpu.async_copy(tab.at[idx_v.at[1]], rows_r, sr)

        def wait_ab(idx_v, rows_l, rows_r, sl, sr):
            pltpu.make_async_copy(tab.at[idx_v.at[0]], rows_l, sl).wait()
            pltpu.make_async_copy(tab.at[idx_v.at[1]], rows_r, sr).wait()

        def fire_scatter(idx_v, scat_v):
            for j in range(BLK // 16):
                scat_v[pl.ds(16 * j, 16)] = idx_v[2, pl.ds(16 * j, 16)]
            pltpu.async_copy(wrow, acc.at[scat_v], ssem, add=True)

        def wait_scatter(scat_v):
            pltpu.make_async_copy(wrow, acc.at[scat_v], ssem).wait()

        fetch(0, idx_a, rows_la, rows_ra, sem_la, sem_ra)

        def pair_body(p, c):
            fetch(2 * p + 1, idx_b, rows_lb, rows_rb, sem_lb, sem_rb)
            wait_ab(idx_a, rows_la, rows_ra, sem_la, sem_ra)

            @pl.when(p > 0)
            def _wsb():
                wait_scatter(scat_b)

            compute(rows_la, rows_ra, idx_a)
            fire_scatter(idx_a, scat_a)

            @pl.when(p < NPAIR - 1)
            def _prefetch_a():
                fetch(2 * p + 2, idx_a, rows_la, rows_ra, sem_la, sem_ra)

            wait_ab(idx_b, rows_lb, rows_rb, sem_lb, sem_rb)
            wait_scatter(scat_a)
            compute(rows_lb, rows_rb, idx_b)
            fire_scatter(idx_b, scat_b)
            return c

        lax.fori_loop(0, NPAIR, pair_body, 0)
        wait_scatter(scat_b)
        plsc.subcore_barrier()

        pltpu.sync_copy(acc.at[pl.ds(sid * 640, 640)],
                        out.at[cid, rel, pl.ds(sid * 640, 640), :])
        plsc.subcore_barrier()
        return carry

    lax.fori_loop(0, 9, rel_body, 0)


def _sc_edge_layer(tab, idxall, att, zrows):
    mesh = plsc.VectorSubcoreMesh(core_axis_name="c", subcore_axis_name="s",
                                  num_cores=2, num_subcores=16)
    return pl.kernel(
        _sc_edge_body,
        out_type=jax.ShapeDtypeStruct((2, 9, ACC_ROWS, WACC), jnp.float32),
        mesh=mesh,
        compiler_params=pltpu.CompilerParams(use_tc_tiling_on_sc=False,
                                             needs_layout_passes=False),
        scratch_types=[
            pltpu.VMEM_SHARED((ACC_ROWS, WACC), jnp.float32),
            pltpu.VMEM((BLK, HID), jnp.float32),
            pltpu.VMEM((BLK, HID), jnp.float32),
            pltpu.VMEM((BLK, HID), jnp.float32),
            pltpu.VMEM((BLK, HID), jnp.float32),
            pltpu.VMEM((BLK, WACC), jnp.float32),
            pltpu.VMEM((ZROWS, WACC), jnp.float32),
            pltpu.VMEM((HID,), jnp.float32),
            pltpu.VMEM((4, BLK), jnp.int32),
            pltpu.VMEM((4, BLK), jnp.int32),
            pltpu.VMEM((BLK,), jnp.int32),
            pltpu.VMEM((BLK,), jnp.int32),
            pltpu.SemaphoreType.DMA,
            pltpu.SemaphoreType.DMA,
            pltpu.SemaphoreType.DMA,
            pltpu.SemaphoreType.DMA,
            pltpu.SemaphoreType.DMA,
            pltpu.SemaphoreType.DMA,
        ],
    )(tab, idxall, att, zrows)


def _edge_indices(eis):
    loops = jnp.arange(N, dtype=jnp.int32)
    padz = jnp.zeros((EPAD - ETOT,), jnp.int32)
    padg = jnp.full((EPAD - ETOT,), GARBAGE, jnp.int32)
    zline = jnp.zeros((EPAD,), jnp.int32)
    rows = []
    for r in RELS:
        ei = eis[r]
        src = jnp.concatenate([ei[0].astype(jnp.int32), loops, padz])
        dst = jnp.concatenate([ei[1].astype(jnp.int32), loops])
        sidx = TIDX[r[0]] * 6 * N + src * 6 + L_SLOT[r]
        didx = TIDX[r[1]] * 6 * N + jnp.concatenate([dst, padz]) * 6 + R_SLOT[r]
        scat = jnp.concatenate([dst, padg])
        quad = jnp.stack([sidx, didx, scat, zline])
        rows.append(quad.reshape(4, NWORK, NCHUNK, BLK).transpose(1, 2, 0, 3))
    return jnp.stack(rows).reshape(9 * NWORK * NCHUNK, 4, BLK)


def _layer_weights(lp):
    wcat, bcat = [], []
    for t in TYPES:
        wcat.append(jnp.concatenate(
            [lp[r]['Wl'] for r in SRC_RELS[t]] + [lp[r]['Wr'] for r in DST_RELS[t]],
            axis=1))
        bcat.append(jnp.concatenate(
            [lp[r]['bl'] for r in SRC_RELS[t]] + [lp[r]['br'] for r in DST_RELS[t]]))
    return jnp.stack(wcat), jnp.stack(bcat).reshape(3, 1, 6 * HID)


def _gnn_layer(x_cat, lp, idxs, zrows, relu):
    wcat, bcat = _layer_weights(lp)
    y = _proj_cat(x_cat, wcat, bcat)
    tab = y.reshape(18 * N, HID)
    att = jnp.concatenate([lp[r]['att'].reshape(HID) for r in RELS])
    partials = _sc_edge_layer(tab, idxs, att, zrows)
    bias_all = jnp.stack([lp[r]['bias'] for r in RELS]).reshape(9, 1, HID)
    return _merge(partials, bias_all, relu)


def kernel(x_tile, x_vertex, x_road, ei_tt, ei_tv, ei_vt, ei_tr, ei_rt,
           ei_vr, ei_rv, ei_vv, ei_rr, params: Any):
    eis = {'tt': ei_tt, 'tv': ei_tv, 'vt': ei_vt, 'tr': ei_tr, 'rt': ei_rt,
           'vr': ei_vr, 'rv': ei_rv, 'vv': ei_vv, 'rr': ei_rr}
    idxs = _edge_indices(eis)
    zrows = jnp.zeros((ZROWS, WACC), jnp.float32)

    x_cat = _proj_in(x_tile, x_vertex, x_road, params)
    x_cat = _gnn_layer(x_cat, params['gnn1'], idxs, zrows, relu=True)
    x_cat = _gnn_layer(x_cat, params['gnn2'], idxs, zrows, relu=False)

    svc, road, rob, g, glob = _tail(x_cat, params)
    return (svc[:, 0], svc[:, 1], road[:, 0], rob[:, 0], glob[0],
            x_cat[0:N], x_cat[N:2 * N], x_cat[2 * N:3 * N], g)


# speedup vs baseline: 26.2855x; 1.0031x over previous
import functools
from typing import Any

import jax
import jax.numpy as jnp
from jax import lax
from jax.experimental import pallas as pl
from jax.experimental.pallas import tpu as pltpu
from jax.experimental.pallas import tpu_sc as plsc

N = 10000
E = 64000
ETOT = E + N
HID = 128
NHEAD = 2
CH = 64
RELS = ('tt', 'tv', 'vt', 'tr', 'rt', 'vr', 'rv', 'vv', 'rr')
TYPES = ('t', 'v', 'r')

NWORK = 32
BLK = 48
NCHUNK = 50
NPAIR = NCHUNK // 2
EPW = BLK * NCHUNK
EPAD = EPW * NWORK
ACC_ROWS = 10240
WACC = 144
GARBAGE = 10008
ZROWS = 32
NZCH = 20

SRC_RELS = {t: [r for r in RELS if r[0] == t] for t in TYPES}
DST_RELS = {t: [r for r in RELS if r[1] == t] for t in TYPES}
L_SLOT = {r: SRC_RELS[r[0]].index(r) for r in RELS}
R_SLOT = {r: 3 + DST_RELS[r[1]].index(r) for r in RELS}
TIDX = {'t': 0, 'v': 1, 'r': 2}
DST_REL_IDS = [[RELS.index(r) for r in DST_RELS[t]] for t in TYPES]

BR = 1000
GRID = N // BR


def _proj_in_body(xt, xv, xr, wt, bt, wv, bv, wr, br, out):
    t = pl.program_id(0)

    def mk(x, w, b):
        def f():
            out[...] = jnp.maximum(
                jnp.dot(x[...], w[...], preferred_element_type=jnp.float32) + b[...], 0.0)
        return f

    branches = [mk(xt, wt, bt), mk(xv, wv, bv), mk(xr, wr, br)]
    lax.switch(t, branches)


def _proj_in(xt, xv, xr, p):
    def w(name):
        return p[name]['W'], p[name]['b'].reshape(1, HID)
    wt, bt = w('in_tile')
    wv, bv = w('in_vertex')
    wr, br = w('in_road')
    row = lambda t, i: (i, 0)
    whole = lambda t, i: (0, 0)
    return pl.pallas_call(
        _proj_in_body,
        grid=(3, GRID),
        in_specs=[
            pl.BlockSpec((BR, 32), row), pl.BlockSpec((BR, 16), row), pl.BlockSpec((BR, 8), row),
            pl.BlockSpec((32, HID), whole), pl.BlockSpec((1, HID), whole),
            pl.BlockSpec((16, HID), whole), pl.BlockSpec((1, HID), whole),
            pl.BlockSpec((8, HID), whole), pl.BlockSpec((1, HID), whole),
        ],
        out_specs=pl.BlockSpec((BR, HID), lambda t, i: (t * GRID + i, 0)),
        out_shape=jax.ShapeDtypeStruct((3 * N, HID), jnp.float32),
    )(xt, xv, xr, wt, bt, wv, bv, wr, br)


def _proj_cat_body(x, w, b, o):
    o[...] = jnp.dot(x[...], w[0], preferred_element_type=jnp.float32) + b[0]


def _proj_cat(x_cat, wcat, bcat):
    return pl.pallas_call(
        _proj_cat_body,
        grid=(3, GRID),
        in_specs=[
            pl.BlockSpec((BR, HID), lambda t, i: (t * GRID + i, 0)),
            pl.BlockSpec((1, HID, 6 * HID), lambda t, i: (t, 0, 0)),
            pl.BlockSpec((1, 1, 6 * HID), lambda t, i: (t, 0, 0)),
        ],
        out_specs=pl.BlockSpec((BR, 6 * HID), lambda t, i: (t * GRID + i, 0)),
        out_shape=jax.ShapeDtypeStruct((3 * N, 6 * HID), jnp.float32),
    )(x_cat, wcat, bcat)


def _merge_body(p00, p01, p02, p10, p11, p12, b0, b1, b2, out, *, relu):
    acc = None
    for (a, b, bias) in ((p00, p10, b0), (p01, p11, b1), (p02, p12, b2)):
        q = a[0, 0] + b[0, 0]
        s0 = q[:, HID:HID + 1] + 1e-30
        s1 = q[:, HID + 1:HID + 2] + 1e-30
        o = jnp.concatenate([q[:, :CH] / s0, q[:, CH:HID] / s1], axis=-1)
        o = o + bias[0]
        acc = o if acc is None else acc + o
    out[...] = jnp.maximum(acc, 0.0) if relu else acc


def _merge(partials, bias_all, relu):
    def rel_of(t, slot):
        v = DST_REL_IDS
        return jnp.where(t == 0, v[0][slot], jnp.where(t == 1, v[1][slot], v[2][slot]))

    specs = []
    for sc in (0, 1):
        for slot in range(3):
            specs.append(pl.BlockSpec(
                (1, 1, BR, WACC),
                functools.partial(lambda t, i, _sc, _k: (_sc, rel_of(t, _k), i, 0),
                                  _sc=sc, _k=slot)))
    for slot in range(3):
        specs.append(pl.BlockSpec(
            (1, 1, HID),
            functools.partial(lambda t, i, _k: (rel_of(t, _k), 0, 0), _k=slot)))
    return pl.pallas_call(
        functools.partial(_merge_body, relu=relu),
        grid=(3, GRID),
        in_specs=specs,
        out_specs=pl.BlockSpec((BR, HID), lambda t, i: (t * GRID + i, 0)),
        out_shape=jax.ShapeDtypeStruct((3 * N, HID), jnp.float32),
    )(partials, partials, partials, partials, partials, partials,
      bias_all, bias_all, bias_all)


def _tail_body(xcat, wsc, bsc, wro, bro, wrb, brb, w1, b1, w2, b2,
               svc, road, rob, g, glob):
    xt = xcat[0:N]
    xv = xcat[N:2 * N]
    xr = xcat[2 * N:3 * N]
    svc[...] = jnp.dot(xv, wsc[...], preferred_element_type=jnp.float32) + bsc[...]
    road[...] = jnp.dot(xr, wro[...], preferred_element_type=jnp.float32) + bro[...]
    rob[...] = jnp.dot(xt, wrb[...], preferred_element_type=jnp.float32) + brb[...]
    mt = jnp.mean(xt, axis=0, keepdims=True)
    mv = jnp.mean(xv, axis=0, keepdims=True)
    mr = jnp.mean(xr, axis=0, keepdims=True)
    gg = jnp.concatenate([mt, mv, mr], axis=-1)
    g[...] = gg
    h = jnp.maximum(jnp.dot(gg, w1[...], preferred_element_type=jnp.float32) + b1[...], 0.0)
    glob[...] = jnp.dot(h, w2[...], preferred_element_type=jnp.float32) + b2[...]


def _tail(xcat, p):
    wsc = jnp.concatenate([p['head_settlement']['W'], p['head_city']['W']], axis=1)
    bsc = jnp.stack([p['head_settlement']['b'][0], p['head_city']['b'][0]]).reshape(1, 2)
    ins = (xcat, wsc, bsc,
           p['head_road']['W'], p['head_road']['b'].reshape(1, 1),
           p['head_robber']['W'], p['head_robber']['b'].reshape(1, 1),
           p['glob1']['W'], p['glob1']['b'].reshape(1, HID),
           p['glob2']['W'], p['glob2']['b'].reshape(1, 2))
    return pl.pallas_call(
        _tail_body,
        out_shape=[
            jax.ShapeDtypeStruct((N, 2), jnp.float32),
            jax.ShapeDtypeStruct((N, 1), jnp.float32),
            jax.ShapeDtypeStruct((N, 1), jnp.float32),
            jax.ShapeDtypeStruct((1, 3 * HID), jnp.float32),
            jax.ShapeDtypeStruct((1, 2), jnp.float32),
        ],
    )(*ins)


def _sc_edge_body(tab, idxall, att, zrows, out,
                  acc, rows_la, rows_ra, rows_lb, rows_rb, wrow, zbuf, att_vb,
                  idx_a, idx_b, scat_a, scat_b,
                  sem_la, sem_ra, sem_lb, sem_rb, zsem, ssem):
    cid = lax.axis_index("c")
    sid = lax.axis_index("s")
    wid = cid * 16 + sid
    io16 = jnp.arange(16, dtype=jnp.int32)

    pltpu.sync_copy(zrows, zbuf)

    def rel_body(rel, carry):
        zds = [pltpu.async_copy(zbuf, acc.at[pl.ds(sid * 640 + z * ZROWS, ZROWS)],
                                zsem) for z in range(NZCH)]
        for zd in zds:
            zd.wait()
        plsc.subcore_barrier()

        pltpu.sync_copy(att.at[pl.ds(rel * HID, HID)], att_vb)
        att_vecs = [att_vb[pl.ds(16 * j, 16)] for j in range(8)]
        chunk0 = (rel * NWORK + wid) * NCHUNK

        def compute(rows_l, rows_r, idx_v):
            def one_edge(e):
                ls = [rows_l[e, pl.ds(16 * j, 16)] for j in range(8)]
                rs = [rows_r[e, pl.ds(16 * j, 16)] for j in range(8)]
                ps = []
                for j in range(8):
                    t = ls[j] + rs[j]
                    t = jnp.maximum(t, 0.2 * t)
                    ps.append(t * att_vecs[j])
                ha = (ps[0] + ps[1]) + (ps[2] + ps[3])
                hb = (ps[4] + ps[5]) + (ps[6] + ps[7])
                ea = jnp.exp(jnp.full((16,), jnp.sum(ha), jnp.float32))
                eb = jnp.exp(jnp.full((16,), jnp.sum(hb), jnp.float32))
                for j in range(4):
                    wrow[e, pl.ds(16 * j, 16)] = ls[j] * ea
                for j in range(4, 8):
                    wrow[e, pl.ds(16 * j, 16)] = ls[j] * eb
                wrow[e, pl.ds(HID, 16)] = (jnp.where(io16 == 0, ea, 0.0)
                                           + jnp.where(io16 == 1, eb, 0.0))

            def edge_body(eq, c):
                for u in range(4):
                    one_edge(eq * 4 + u)
                return c
            lax.fori_loop(0, BLK // 4, edge_body, 0)

        def fetch(ch, idx_v, rows_l, rows_r, sl, sr):
            pltpu.sync_copy(idxall.at[chunk0 + ch], idx_v)
            pltpu.async_copy(tab.at[idx_v.at[0]], rows_l, sl)
            pltpu.async_copy(tab.at[idx_v.at[1]], rows_r, sr)

        def wait_ab(idx_v, rows_l, rows_r, sl, sr):
            pltpu.make_async_copy(tab.at[idx_v.at[0]], rows_l, sl).wait()
            pltpu.make_async_copy(tab.at[idx_v.at[1]], rows_r, sr).wait()

        def fire_scatter(idx_v, scat_v):
            for j in range(BLK // 16):
                scat_v[pl.ds(16 * j, 16)] = idx_v[2, pl.ds(16 * j, 16)]
            pltpu.async_copy(wrow, acc.at[scat_v], ssem, add=True)

        def wait_scatter(scat_v):
            pltpu.make_async_copy(wrow, acc.at[scat_v], ssem).wait()

        fetch(0, idx_a, rows_la, rows_ra, sem_la, sem_ra)

        def pair_body(p, c):
            fetch(2 * p + 1, idx_b, rows_lb, rows_rb, sem_lb, sem_rb)
            wait_ab(idx_a, rows_la, rows_ra, sem_la, sem_ra)

            @pl.when(p > 0)
            def _wsb():
                wait_scatter(scat_b)

            compute(rows_la, rows_ra, idx_a)
            fire_scatter(idx_a, scat_a)

            @pl.when(p < NPAIR - 1)
            def _prefetch_a():
                fetch(2 * p + 2, idx_a, rows_la, rows_ra, sem_la, sem_ra)

            wait_ab(idx_b, rows_lb, rows_rb, sem_lb, sem_rb)
            wait_scatter(scat_a)
            compute(rows_lb, rows_rb, idx_b)
            fire_scatter(idx_b, scat_b)
            return c

        lax.fori_loop(0, NPAIR, pair_body, 0)
        wait_scatter(scat_b)
        plsc.subcore_barrier()

        pltpu.sync_copy(acc.at[pl.ds(sid * 640, 640)],
                        out.at[cid, rel, pl.ds(sid * 640, 640), :])
        plsc.subcore_barrier()
        return carry

    lax.fori_loop(0, 9, rel_body, 0)


def _sc_edge_layer(tab, idxall, att, zrows):
    mesh = plsc.VectorSubcoreMesh(core_axis_name="c", subcore_axis_name="s",
                                  num_cores=2, num_subcores=16)
    return pl.kernel(
        _sc_edge_body,
        out_type=jax.ShapeDtypeStruct((2, 9, ACC_ROWS, WACC), jnp.float32),
        mesh=mesh,
        compiler_params=pltpu.CompilerParams(use_tc_tiling_on_sc=False,
                                             needs_layout_passes=False),
        scratch_types=[
            pltpu.VMEM_SHARED((ACC_ROWS, WACC), jnp.float32),
            pltpu.VMEM((BLK, HID), jnp.float32),
            pltpu.VMEM((BLK, HID), jnp.float32),
            pltpu.VMEM((BLK, HID), jnp.float32),
            pltpu.VMEM((BLK, HID), jnp.float32),
            pltpu.VMEM((BLK, WACC), jnp.float32),
            pltpu.VMEM((ZROWS, WACC), jnp.float32),
            pltpu.VMEM((HID,), jnp.float32),
            pltpu.VMEM((4, BLK), jnp.int32),
            pltpu.VMEM((4, BLK), jnp.int32),
            pltpu.VMEM((BLK,), jnp.int32),
            pltpu.VMEM((BLK,), jnp.int32),
            pltpu.SemaphoreType.DMA,
            pltpu.SemaphoreType.DMA,
            pltpu.SemaphoreType.DMA,
            pltpu.SemaphoreType.DMA,
            pltpu.SemaphoreType.DMA,
            pltpu.SemaphoreType.DMA,
        ],
    )(tab, idxall, att, zrows)


def _edge_indices(eis):
    loops = jnp.arange(N, dtype=jnp.int32)
    padz = jnp.zeros((EPAD - ETOT,), jnp.int32)
    padg = jnp.full((EPAD - ETOT,), GARBAGE, jnp.int32)
    zline = jnp.zeros((EPAD,), jnp.int32)
    rows = []
    for r in RELS:
        ei = eis[r]
        src = jnp.concatenate([ei[0].astype(jnp.int32), loops, padz])
        dst = jnp.concatenate([ei[1].astype(jnp.int32), loops])
        sidx = TIDX[r[0]] * 6 * N + src * 6 + L_SLOT[r]
        didx = TIDX[r[1]] * 6 * N + jnp.concatenate([dst, padz]) * 6 + R_SLOT[r]
        scat = jnp.concatenate([dst, padg])
        quad = jnp.stack([sidx, didx, scat, zline])
        rows.append(quad.reshape(4, NWORK, NCHUNK, BLK).transpose(1, 2, 0, 3))
    return jnp.stack(rows).reshape(9 * NWORK * NCHUNK, 4, BLK)


def _layer_weights(lp):
    wcat, bcat = [], []
    for t in TYPES:
        wcat.append(jnp.concatenate(
            [lp[r]['Wl'] for r in SRC_RELS[t]] + [lp[r]['Wr'] for r in DST_RELS[t]],
            axis=1))
        bcat.append(jnp.concatenate(
            [lp[r]['bl'] for r in SRC_RELS[t]] + [lp[r]['br'] for r in DST_RELS[t]]))
    return jnp.stack(wcat), jnp.stack(bcat).reshape(3, 1, 6 * HID)


def _gnn_layer(x_cat, lp, idxs, zrows, relu):
    wcat, bcat = _layer_weights(lp)
    y = _proj_cat(x_cat, wcat, bcat)
    tab = y.reshape(18 * N, HID)
    att = jnp.concatenate([lp[r]['att'].reshape(HID) for r in RELS])
    partials = _sc_edge_layer(tab, idxs, att, zrows)
    bias_all = jnp.stack([lp[r]['bias'] for r in RELS]).reshape(9, 1, HID)
    return _merge(partials, bias_all, relu)


def kernel(x_tile, x_vertex, x_road, ei_tt, ei_tv, ei_vt, ei_tr, ei_rt,
           ei_vr, ei_rv, ei_vv, ei_rr, params: Any):
    eis = {'tt': ei_tt, 'tv': ei_tv, 'vt': ei_vt, 'tr': ei_tr, 'rt': ei_rt,
           'vr': ei_vr, 'rv': ei_rv, 'vv': ei_vv, 'rr': ei_rr}
    idxs = _edge_indices(eis)
    zrows = jnp.zeros((ZROWS, WACC), jnp.float32)

    x_cat = _proj_in(x_tile, x_vertex, x_road, params)
    x_cat = _gnn_layer(x_cat, params['gnn1'], idxs, zrows, relu=True)
    x_cat = _gnn_layer(x_cat, params['gnn2'], idxs, zrows, relu=False)

    svc, road, rob, g, glob = _tail(x_cat, params)
    return (svc[:, 0], svc[:, 1], road[:, 0], rob[:, 0], glob[0],
            x_cat[0:N], x_cat[N:2 * N], x_cat[2 * N:3 * N], g)
